# baseline (speedup 1.0000x reference)
import functools

import jax
import jax.numpy as jnp
from jax import lax
from jax.experimental import pallas as pl
from jax.experimental.pallas import tpu as pltpu
from jax.experimental.pallas import tpu_sc as plsc

N = 10000
IN_DIM = 128
OUT_DIM = 16
E = 320000

NC = 2
NS = 16
L = 16
NW = NC * NS

N_PAD = 10240
RPT = N_PAD // NS
CHUNK = 128
K = 80
EPT = K * CHUNK
E_PAD = EPT * NW
KD = E_PAD // (NS * CHUNK)

NB = 8
KB = K // NB
KDB = KD // NB

_mesh = plsc.VectorSubcoreMesh(core_axis_name="c", subcore_axis_name="s")


def _rsqrt16(d):
    i = plsc.bitcast(d, jnp.int32)
    i = jnp.int32(0x5F3759DF) - lax.shift_right_logical(i, 1)
    y = plsc.bitcast(i, jnp.float32)
    for _ in range(3):
        y = y * (1.5 - 0.5 * d * y * y)
    return y


@functools.partial(
    pl.kernel,
    out_type=(jax.ShapeDtypeStruct((NC, N_PAD, OUT_DIM), jnp.float32),
              jax.ShapeDtypeStruct((NC, N_PAD), jnp.float32)),
    mesh=_mesh,
    scratch_types=[
        pltpu.VMEM((KD, CHUNK), jnp.int32),
        pltpu.VMEM((K, CHUNK), jnp.int32),
        pltpu.VMEM((K, CHUNK), jnp.int32),
        pltpu.VMEM((CHUNK,), jnp.float32),
        pltpu.VMEM((RPT,), jnp.float32),
        pltpu.VMEM((RPT,), jnp.float32),
        pltpu.VMEM((RPT, OUT_DIM), jnp.float32),
        pltpu.VMEM((NB, CHUNK, OUT_DIM), jnp.float32),
        pltpu.VMEM_SHARED((N_PAD,), jnp.float32),
        pltpu.VMEM_SHARED((N_PAD, OUT_DIM), jnp.float32),
        pltpu.VMEM_SHARED((N_PAD, OUT_DIM), jnp.float32),
    ] + [pltpu.SemaphoreType.DMA] * (2 * NB),
    compiler_params=pltpu.CompilerParams(use_tc_tiling_on_sc=False,
                                         needs_layout_passes=False),
)
def _mega_kernel(src_hbm, dst_hbm, h_hbm, accp_hbm, degp_hbm,
                 didxd_v, sidx_v, didx_v, ones_v, degv, dinvv, hv, rows_v,
                 deg_sh, g_sh, acc_sh, *sems):
    gsem, ssem = sems[:NB], sems[NB:]
    c = lax.axis_index("c")
    s = lax.axis_index("s")
    wid = s * NC + c
    row0 = s * RPT
    one = jnp.ones((L,), jnp.float32)
    zero = jnp.zeros((L,), jnp.float32)

    for i in range(CHUNK // L):
        ones_v[pl.ds(i * L, L)] = one
    for i in range(RPT // L):
        degv[pl.ds(i * L, L)] = zero
    pltpu.sync_copy(degv, deg_sh.at[pl.ds(row0, RPT)])
    for i in range(CHUNK):
        rows_v[0, i, :] = zero
    for t in range(RPT // CHUNK):
        pltpu.sync_copy(rows_v.at[0], acc_sh.at[pl.ds(row0 + t * CHUNK, CHUNK)])
    pltpu.sync_copy(dst_hbm.at[2 * s], didxd_v.at[pl.ds(0, K)])
    pltpu.sync_copy(dst_hbm.at[2 * s + 1], didxd_v.at[pl.ds(K, K)])
    pltpu.sync_copy(src_hbm.at[wid], sidx_v)
    pltpu.sync_copy(dst_hbm.at[wid], didx_v)
    pltpu.sync_copy(h_hbm.at[pl.ds(row0, RPT)], hv)
    plsc.subcore_barrier()

    for b in range(NB):
        pltpu.async_copy(ones_v, deg_sh.at[didxd_v.at[b]], ssem[b], add=True)

    def dbody(t, carry):
        for b in range(NB):
            j = t * NB + b
            pltpu.make_async_copy(ones_v, deg_sh.at[didxd_v.at[j]],
                                  ssem[b]).wait()
            pltpu.async_copy(ones_v, deg_sh.at[didxd_v.at[j + NB]], ssem[b],
                             add=True)
        return carry

    lax.fori_loop(0, KDB - 1, dbody, 0)
    for b in range(NB):
        j = (KDB - 1) * NB + b
        pltpu.make_async_copy(ones_v, deg_sh.at[didxd_v.at[j]], ssem[b]).wait()
    plsc.subcore_barrier()

    pltpu.sync_copy(deg_sh.at[pl.ds(row0, RPT)], degv)

    def vbody(i, carry):
        d = degv[pl.ds(i * L, L)] + 1.0
        dinvv[pl.ds(i * L, L)] = _rsqrt16(d)
        return carry

    lax.fori_loop(0, RPT // L, vbody, 0)

    def gbody(i, carry):
        dv = dinvv[pl.ds(i * L, L)]
        for bl in range(L):
            r = i * L + bl
            hv[r, :] = hv[r, :] * dv[bl]
        return carry

    lax.fori_loop(0, RPT // L, gbody, 0)
    pltpu.sync_copy(hv, g_sh.at[pl.ds(row0, RPT)])
    pltpu.sync_copy(degv, degp_hbm.at[c].at[pl.ds(row0, RPT)])
    plsc.subcore_barrier()

    for b in range(NB):
        pltpu.async_copy(g_sh.at[sidx_v.at[b]], rows_v.at[b], gsem[b])

    def abody(t, carry):
        j0 = t * NB
        for b in range(NB):
            j = j0 + b
            pltpu.make_async_copy(g_sh.at[sidx_v.at[j]], rows_v.at[b],
                                  gsem[b]).wait()
            pltpu.async_copy(rows_v.at[b], acc_sh.at[didx_v.at[j]], ssem[b],
                             add=True)
        for b in range(NB):
            j = j0 + b
            pltpu.make_async_copy(rows_v.at[b], acc_sh.at[didx_v.at[j]],
                                  ssem[b]).wait()
            pltpu.async_copy(g_sh.at[sidx_v.at[j + NB]], rows_v.at[b],
                             gsem[b])
        return carry

    lax.fori_loop(0, KB - 1, abody, 0)
    for b in range(NB):
        j = (KB - 1) * NB + b
        pltpu.make_async_copy(g_sh.at[sidx_v.at[j]], rows_v.at[b],
                              gsem[b]).wait()
        pltpu.async_copy(rows_v.at[b], acc_sh.at[didx_v.at[j]], ssem[b],
                         add=True)
    for b in range(NB):
        j = (KB - 1) * NB + b
        pltpu.make_async_copy(rows_v.at[b], acc_sh.at[didx_v.at[j]],
                              ssem[b]).wait()
    plsc.subcore_barrier()
    pltpu.sync_copy(acc_sh.at[pl.ds(row0, RPT)],
                    accp_hbm.at[c].at[pl.ds(row0, RPT)])


def _linear_body(x_ref, w_ref, h_ref):
    h = lax.dot_general(x_ref[...], w_ref[...],
                        (((1,), (1,)), ((), ())),
                        preferred_element_type=jnp.float32)
    h_ref[:N, :] = h
    h_ref[N:, :] = jnp.zeros((N_PAD - N, OUT_DIM), jnp.float32)


def _combine_body(accp_ref, h_ref, degp_ref, b_ref, out_ref):
    deg = degp_ref[0, :N] + 1.0
    dinv = lax.rsqrt(deg)
    g = h_ref[:N, :] * dinv[:, None]
    acc = accp_ref[0, :N, :] + accp_ref[1, :N, :] + g
    out_ref[...] = acc * dinv[:, None] + b_ref[...]


def kernel(x, edge_index, W, b):
    src = edge_index[0].astype(jnp.int32)
    dst = edge_index[1].astype(jnp.int32)
    pad = E_PAD - E
    ar = jnp.arange(pad, dtype=jnp.int32)
    src_f = jnp.concatenate([src, ar % 211])
    dst_f = jnp.concatenate([dst, N + (ar % (N_PAD - N))])
    src_p = src_f.reshape(NW, K, CHUNK)
    dst_p = dst_f.reshape(NW, K, CHUNK)

    h = pl.pallas_call(
        _linear_body,
        out_shape=jax.ShapeDtypeStruct((N_PAD, OUT_DIM), jnp.float32),
    )(x, W)
    accp, degp = _mega_kernel(src_p, dst_p, h)
    out = pl.pallas_call(
        _combine_body,
        out_shape=jax.ShapeDtypeStruct((N, OUT_DIM), jnp.float32),
    )(accp, h, degp, b.reshape(1, OUT_DIM))
    return out


# speedup vs baseline: 1.1656x; 1.1656x over previous
import functools

import jax
import jax.numpy as jnp
from jax import lax
from jax.experimental import pallas as pl
from jax.experimental.pallas import tpu as pltpu
from jax.experimental.pallas import tpu_sc as plsc

N = 10000
IN_DIM = 128
OUT_DIM = 16
E = 320000

NC = 2
NS = 16
L = 16
NW = NC * NS

N_PAD = 10240
RPT = N_PAD // NS
RPW = N_PAD // NW
CHUNK = 80
K = 125
EPT = K * CHUNK
KD = 2 * K

NB = 5
KB = K // NB
KDB = KD // NB

_mesh = plsc.VectorSubcoreMesh(core_axis_name="c", subcore_axis_name="s")


def _rsqrt16(d):
    i = plsc.bitcast(d, jnp.int32)
    i = jnp.int32(0x5F3759DF) - lax.shift_right_logical(i, 1)
    y = plsc.bitcast(i, jnp.float32)
    for _ in range(3):
        y = y * (1.5 - 0.5 * d * y * y)
    return y


@functools.partial(
    pl.kernel,
    out_type=(jax.ShapeDtypeStruct((NC, N_PAD, OUT_DIM), jnp.float32),
              jax.ShapeDtypeStruct((NC, N_PAD), jnp.float32)),
    mesh=_mesh,
    scratch_types=[
        pltpu.VMEM((KD, CHUNK), jnp.int32),
        pltpu.VMEM((K, CHUNK), jnp.int32),
        pltpu.VMEM((K, CHUNK), jnp.int32),
        pltpu.VMEM((CHUNK,), jnp.float32),
        pltpu.VMEM((RPT,), jnp.float32),
        pltpu.VMEM((RPT,), jnp.float32),
        pltpu.VMEM((RPT, OUT_DIM), jnp.float32),
        pltpu.VMEM((NB, CHUNK, OUT_DIM), jnp.float32),
        pltpu.VMEM_SHARED((N_PAD,), jnp.float32),
        pltpu.VMEM_SHARED((N_PAD, OUT_DIM), jnp.float32),
        pltpu.VMEM_SHARED((N_PAD, OUT_DIM), jnp.float32),
    ] + [pltpu.SemaphoreType.DMA] * (2 * NB),
    compiler_params=pltpu.CompilerParams(use_tc_tiling_on_sc=False,
                                         needs_layout_passes=False),
)
def _mega_kernel(ei_hbm, h_hbm, accp_hbm, degp_hbm,
                 didxd_v, sidx_v, didx_v, ones_v, degv, dinvv, hv, rows_v,
                 deg_sh, g_sh, acc_sh, *sems):
    gsem, ssem = sems[:NB], sems[NB:]
    c = lax.axis_index("c")
    s = lax.axis_index("s")
    wid = s * NC + c
    row0 = s * RPT
    one = jnp.ones((L,), jnp.float32)
    zero = jnp.zeros((L,), jnp.float32)

    for i in range(CHUNK // L):
        ones_v[pl.ds(i * L, L)] = one
    for i in range(RPT // L):
        degv[pl.ds(i * L, L)] = zero
    pltpu.sync_copy(degv, deg_sh.at[pl.ds(row0, RPT)])
    for i in range(CHUNK):
        rows_v[0, i, :] = zero
    for t in range(RPT // CHUNK):
        pltpu.sync_copy(rows_v.at[0], acc_sh.at[pl.ds(row0 + t * CHUNK, CHUNK)])
    pltpu.sync_copy(ei_hbm.at[1].at[2 * s], didxd_v.at[pl.ds(0, K)])
    pltpu.sync_copy(ei_hbm.at[1].at[2 * s + 1], didxd_v.at[pl.ds(K, K)])
    pltpu.sync_copy(ei_hbm.at[0].at[wid], sidx_v)
    pltpu.sync_copy(ei_hbm.at[1].at[wid], didx_v)
    pltpu.sync_copy(h_hbm.at[pl.ds(row0, RPT)], hv)
    plsc.subcore_barrier()

    for b in range(NB):
        pltpu.async_copy(ones_v, deg_sh.at[didxd_v.at[b]], ssem[b], add=True)

    def dbody(t, carry):
        for b in range(NB):
            j = t * NB + b
            pltpu.make_async_copy(ones_v, deg_sh.at[didxd_v.at[j]],
                                  ssem[b]).wait()
            pltpu.async_copy(ones_v, deg_sh.at[didxd_v.at[j + NB]], ssem[b],
                             add=True)
        return carry

    lax.fori_loop(0, KDB - 1, dbody, 0)
    for b in range(NB):
        j = (KDB - 1) * NB + b
        pltpu.make_async_copy(ones_v, deg_sh.at[didxd_v.at[j]], ssem[b]).wait()
    plsc.subcore_barrier()

    pltpu.sync_copy(deg_sh.at[pl.ds(row0, RPT)], degv)

    def vbody(i, carry):
        d = degv[pl.ds(i * L, L)] + 1.0
        dinvv[pl.ds(i * L, L)] = _rsqrt16(d)
        return carry

    lax.fori_loop(0, RPT // L, vbody, 0)

    def gbody(i, carry):
        dv = dinvv[pl.ds(i * L, L)]
        for bl in range(L):
            r = i * L + bl
            hv[r, :] = hv[r, :] * dv[bl]
        return carry

    lax.fori_loop(0, RPT // L, gbody, 0)
    pltpu.sync_copy(hv, g_sh.at[pl.ds(row0, RPT)])
    pltpu.sync_copy(degv, degp_hbm.at[c].at[pl.ds(row0, RPT)])
    plsc.subcore_barrier()

    for b in range(NB):
        pltpu.async_copy(g_sh.at[sidx_v.at[b]], rows_v.at[b], gsem[b])

    def abody(t, carry):
        j0 = t * NB
        for b in range(NB):
            j = j0 + b
            pltpu.make_async_copy(g_sh.at[sidx_v.at[j]], rows_v.at[b],
                                  gsem[b]).wait()
            pltpu.async_copy(rows_v.at[b], acc_sh.at[didx_v.at[j]], ssem[b],
                             add=True)
        for b in range(NB):
            j = j0 + b
            pltpu.make_async_copy(rows_v.at[b], acc_sh.at[didx_v.at[j]],
                                  ssem[b]).wait()
            pltpu.async_copy(g_sh.at[sidx_v.at[j + NB]], rows_v.at[b],
                             gsem[b])
        return carry

    lax.fori_loop(0, KB - 1, abody, 0)
    for b in range(NB):
        j = (KB - 1) * NB + b
        pltpu.make_async_copy(g_sh.at[sidx_v.at[j]], rows_v.at[b],
                              gsem[b]).wait()
        pltpu.async_copy(rows_v.at[b], acc_sh.at[didx_v.at[j]], ssem[b],
                         add=True)
    for b in range(NB):
        j = (KB - 1) * NB + b
        pltpu.make_async_copy(rows_v.at[b], acc_sh.at[didx_v.at[j]],
                              ssem[b]).wait()
    plsc.subcore_barrier()
    pltpu.sync_copy(acc_sh.at[pl.ds(row0, RPT)],
                    accp_hbm.at[c].at[pl.ds(row0, RPT)])


@functools.partial(
    pl.kernel,
    out_type=jax.ShapeDtypeStruct((N_PAD, OUT_DIM), jnp.float32),
    mesh=_mesh,
    scratch_types=[
        pltpu.VMEM((RPW, OUT_DIM), jnp.float32),
        pltpu.VMEM((RPW, OUT_DIM), jnp.float32),
        pltpu.VMEM((RPW, OUT_DIM), jnp.float32),
        pltpu.VMEM((RPW,), jnp.float32),
        pltpu.VMEM((OUT_DIM,), jnp.float32),
    ],
    compiler_params=pltpu.CompilerParams(use_tc_tiling_on_sc=False,
                                         needs_layout_passes=False),
)
def _combine_kernel(accp_hbm, h_hbm, degp_hbm, b_hbm, out_hbm,
                    a0v, a1v, hv, degv, bv):
    c = lax.axis_index("c")
    s = lax.axis_index("s")
    wid = s * NC + c
    r0 = wid * RPW
    pltpu.sync_copy(accp_hbm.at[0].at[pl.ds(r0, RPW)], a0v)
    pltpu.sync_copy(accp_hbm.at[1].at[pl.ds(r0, RPW)], a1v)
    pltpu.sync_copy(h_hbm.at[pl.ds(r0, RPW)], hv)
    pltpu.sync_copy(degp_hbm.at[0].at[pl.ds(r0, RPW)], degv)
    pltpu.sync_copy(b_hbm, bv)
    bb = bv[...]

    def body(i, carry):
        dv = _rsqrt16(degv[pl.ds(i * L, L)] + 1.0)
        for bl in range(L):
            r = i * L + bl
            hv[r, :] = (a0v[r, :] + a1v[r, :] + hv[r, :] * dv[bl]) * dv[bl] + bb
        return carry

    lax.fori_loop(0, RPW // L, body, 0)
    pltpu.sync_copy(hv, out_hbm.at[pl.ds(r0, RPW)])


def _linear_body(x_ref, w_ref, h_ref):
    h = lax.dot_general(x_ref[...], w_ref[...],
                        (((1,), (1,)), ((), ())),
                        preferred_element_type=jnp.float32)
    h_ref[:N, :] = h
    h_ref[N:, :] = jnp.zeros((N_PAD - N, OUT_DIM), jnp.float32)


def kernel(x, edge_index, W, b):
    ei = edge_index.astype(jnp.int32).reshape(2, NW, K, CHUNK)
    h = pl.pallas_call(
        _linear_body,
        out_shape=jax.ShapeDtypeStruct((N_PAD, OUT_DIM), jnp.float32),
    )(x, W)
    accp, degp = _mega_kernel(ei, h)
    out_full = _combine_kernel(accp, h, degp, b)
    return out_full[:N]


# speedup vs baseline: 1.2607x; 1.0816x over previous
import functools

import jax
import jax.numpy as jnp
from jax import lax
from jax.experimental import pallas as pl
from jax.experimental.pallas import tpu as pltpu
from jax.experimental.pallas import tpu_sc as plsc

N = 10000
IN_DIM = 128
OUT_DIM = 16
E = 320000

NC = 2
NS = 16
L = 16
NW = NC * NS

N_PAD = 10240
RPT = N_PAD // NS
RPW = N_PAD // NW
CHUNK = 80
K = 125
EPT = K * CHUNK
KD = 2 * K

NB = 5
KB = K // NB
KDB = KD // NB

_mesh = plsc.VectorSubcoreMesh(core_axis_name="c", subcore_axis_name="s")


def _rsqrt16(d):
    i = plsc.bitcast(d, jnp.int32)
    i = jnp.int32(0x5F3759DF) - lax.shift_right_logical(i, 1)
    y = plsc.bitcast(i, jnp.float32)
    for _ in range(3):
        y = y * (1.5 - 0.5 * d * y * y)
    return y


@functools.partial(
    pl.kernel,
    out_type=jax.ShapeDtypeStruct((NC, N_PAD), jnp.float32),
    mesh=_mesh,
    scratch_types=[
        pltpu.VMEM((K, CHUNK), jnp.int32),
        pltpu.VMEM((CHUNK,), jnp.float32),
        pltpu.VMEM((RPT,), jnp.float32),
        pltpu.VMEM_SHARED((N_PAD,), jnp.float32),
    ] + [pltpu.SemaphoreType.DMA] * NB,
    compiler_params=pltpu.CompilerParams(use_tc_tiling_on_sc=False,
                                         needs_layout_passes=False),
)
def _deg_kernel(ei_hbm, degp_hbm, didx_v, ones_v, zb_v, deg_sh, *ssem):
    c = lax.axis_index("c")
    s = lax.axis_index("s")
    wid = s * NC + c
    row0 = s * RPT
    one = jnp.ones((L,), jnp.float32)
    zero = jnp.zeros((L,), jnp.float32)
    for i in range(CHUNK // L):
        ones_v[pl.ds(i * L, L)] = one
    for i in range(RPT // L):
        zb_v[pl.ds(i * L, L)] = zero
    pltpu.sync_copy(zb_v, deg_sh.at[pl.ds(row0, RPT)])
    pltpu.sync_copy(ei_hbm.at[1].at[wid], didx_v)
    plsc.subcore_barrier()

    for b in range(NB):
        pltpu.async_copy(ones_v, deg_sh.at[didx_v.at[b]], ssem[b], add=True)

    def dbody(t, carry):
        for b in range(NB):
            j = t * NB + b
            pltpu.make_async_copy(ones_v, deg_sh.at[didx_v.at[j]],
                                  ssem[b]).wait()
            pltpu.async_copy(ones_v, deg_sh.at[didx_v.at[j + NB]], ssem[b],
                             add=True)
        return carry

    lax.fori_loop(0, KB - 1, dbody, 0)
    for b in range(NB):
        j = (KB - 1) * NB + b
        pltpu.make_async_copy(ones_v, deg_sh.at[didx_v.at[j]], ssem[b]).wait()
    plsc.subcore_barrier()
    pltpu.sync_copy(deg_sh.at[pl.ds(row0, RPT)],
                    degp_hbm.at[c].at[pl.ds(row0, RPT)])


@functools.partial(
    pl.kernel,
    out_type=jax.ShapeDtypeStruct((NC, N_PAD, OUT_DIM), jnp.float32),
    mesh=_mesh,
    scratch_types=[
        pltpu.VMEM((K, CHUNK), jnp.int32),
        pltpu.VMEM((K, CHUNK), jnp.int32),
        pltpu.VMEM((RPT,), jnp.float32),
        pltpu.VMEM((RPT,), jnp.float32),
        pltpu.VMEM((RPT, OUT_DIM), jnp.float32),
        pltpu.VMEM((NB, CHUNK, OUT_DIM), jnp.float32),
        pltpu.VMEM_SHARED((N_PAD, OUT_DIM), jnp.float32),
        pltpu.VMEM_SHARED((N_PAD, OUT_DIM), jnp.float32),
    ] + [pltpu.SemaphoreType.DMA] * (2 * NB),
    compiler_params=pltpu.CompilerParams(use_tc_tiling_on_sc=False,
                                         needs_layout_passes=False),
)
def _agg_kernel(ei_hbm, h_hbm, degp_hbm, accp_hbm,
                sidx_v, didx_v, degv, dinvv, hv, rows_v,
                g_sh, acc_sh, *sems):
    gsem, ssem = sems[:NB], sems[NB:]
    c = lax.axis_index("c")
    s = lax.axis_index("s")
    wid = s * NC + c
    row0 = s * RPT
    zero = jnp.zeros((L,), jnp.float32)

    for i in range(CHUNK):
        rows_v[0, i, :] = zero
    for t in range(RPT // CHUNK):
        pltpu.sync_copy(rows_v.at[0], acc_sh.at[pl.ds(row0 + t * CHUNK, CHUNK)])
    pltpu.sync_copy(ei_hbm.at[0].at[wid], sidx_v)
    pltpu.sync_copy(ei_hbm.at[1].at[wid], didx_v)
    pltpu.sync_copy(h_hbm.at[pl.ds(row0, RPT)], hv)
    pltpu.sync_copy(degp_hbm.at[0].at[pl.ds(row0, RPT)], degv)
    pltpu.sync_copy(degp_hbm.at[1].at[pl.ds(row0, RPT)], dinvv)

    def vbody(i, carry):
        d = degv[pl.ds(i * L, L)] + dinvv[pl.ds(i * L, L)] + 1.0
        dinvv[pl.ds(i * L, L)] = _rsqrt16(d)
        return carry

    lax.fori_loop(0, RPT // L, vbody, 0)

    def gbody(i, carry):
        dv = dinvv[pl.ds(i * L, L)]
        for bl in range(L):
            r = i * L + bl
            hv[r, :] = hv[r, :] * dv[bl]
        return carry

    lax.fori_loop(0, RPT // L, gbody, 0)
    pltpu.sync_copy(hv, g_sh.at[pl.ds(row0, RPT)])
    plsc.subcore_barrier()

    for b in range(NB):
        pltpu.async_copy(g_sh.at[sidx_v.at[b]], rows_v.at[b], gsem[b])

    def abody(t, carry):
        j0 = t * NB
        for b in range(NB):
            j = j0 + b
            pltpu.make_async_copy(g_sh.at[sidx_v.at[j]], rows_v.at[b],
                                  gsem[b]).wait()
            pltpu.async_copy(rows_v.at[b], acc_sh.at[didx_v.at[j]], ssem[b],
                             add=True)
        for b in range(NB):
            j = j0 + b
            pltpu.make_async_copy(rows_v.at[b], acc_sh.at[didx_v.at[j]],
                                  ssem[b]).wait()
            pltpu.async_copy(g_sh.at[sidx_v.at[j + NB]], rows_v.at[b],
                             gsem[b])
        return carry

    lax.fori_loop(0, KB - 1, abody, 0)
    for b in range(NB):
        j = (KB - 1) * NB + b
        pltpu.make_async_copy(g_sh.at[sidx_v.at[j]], rows_v.at[b],
                              gsem[b]).wait()
        pltpu.async_copy(rows_v.at[b], acc_sh.at[didx_v.at[j]], ssem[b],
                         add=True)
    for b in range(NB):
        j = (KB - 1) * NB + b
        pltpu.make_async_copy(rows_v.at[b], acc_sh.at[didx_v.at[j]],
                              ssem[b]).wait()
    plsc.subcore_barrier()
    pltpu.sync_copy(acc_sh.at[pl.ds(row0, RPT)],
                    accp_hbm.at[c].at[pl.ds(row0, RPT)])


@functools.partial(
    pl.kernel,
    out_type=jax.ShapeDtypeStruct((N_PAD, OUT_DIM), jnp.float32),
    mesh=_mesh,
    scratch_types=[
        pltpu.VMEM((RPW, OUT_DIM), jnp.float32),
        pltpu.VMEM((RPW, OUT_DIM), jnp.float32),
        pltpu.VMEM((RPW, OUT_DIM), jnp.float32),
        pltpu.VMEM((RPW,), jnp.float32),
        pltpu.VMEM((RPW,), jnp.float32),
        pltpu.VMEM((OUT_DIM,), jnp.float32),
    ],
    compiler_params=pltpu.CompilerParams(use_tc_tiling_on_sc=False,
                                         needs_layout_passes=False),
)
def _combine_kernel(accp_hbm, h_hbm, degp_hbm, b_hbm, out_hbm,
                    a0v, a1v, hv, degv, degv1, bv):
    c = lax.axis_index("c")
    s = lax.axis_index("s")
    wid = s * NC + c
    r0 = wid * RPW
    pltpu.sync_copy(accp_hbm.at[0].at[pl.ds(r0, RPW)], a0v)
    pltpu.sync_copy(accp_hbm.at[1].at[pl.ds(r0, RPW)], a1v)
    pltpu.sync_copy(h_hbm.at[pl.ds(r0, RPW)], hv)
    pltpu.sync_copy(degp_hbm.at[0].at[pl.ds(r0, RPW)], degv)
    pltpu.sync_copy(degp_hbm.at[1].at[pl.ds(r0, RPW)], degv1)
    pltpu.sync_copy(b_hbm, bv)
    bb = bv[...]

    def body(i, carry):
        dv = _rsqrt16(degv[pl.ds(i * L, L)] + degv1[pl.ds(i * L, L)] + 1.0)
        for bl in range(L):
            r = i * L + bl
            hv[r, :] = (a0v[r, :] + a1v[r, :] + hv[r, :] * dv[bl]) * dv[bl] + bb
        return carry

    lax.fori_loop(0, RPW // L, body, 0)
    pltpu.sync_copy(hv, out_hbm.at[pl.ds(r0, RPW)])


def _linear_body(x_ref, w_ref, h_ref):
    h = lax.dot_general(x_ref[...], w_ref[...],
                        (((1,), (1,)), ((), ())),
                        preferred_element_type=jnp.float32)
    h_ref[:N, :] = h
    h_ref[N:, :] = jnp.zeros((N_PAD - N, OUT_DIM), jnp.float32)


def kernel(x, edge_index, W, b):
    ei = edge_index.astype(jnp.int32).reshape(2, NW, K, CHUNK)
    degp = _deg_kernel(ei)
    h = pl.pallas_call(
        _linear_body,
        out_shape=jax.ShapeDtypeStruct((N_PAD, OUT_DIM), jnp.float32),
    )(x, W)
    accp = _agg_kernel(ei, h, degp)
    out_full = _combine_kernel(accp, h, degp, b)
    return out_full[:N]
